# baseline (speedup 1.0000x reference)
import jax
import jax.numpy as jnp
from jax import lax
from jax.experimental import pallas as pl
from jax.experimental.pallas import tpu as pltpu
from jax.experimental.pallas import tpu_sc as plsc

N_NODES = 10000
N_EDGES = 160000
FEAT = 256
HALF = 128
N_GRAPHS = 64

NUM_CORES = 2
NUM_SUBCORES = 16
LANES = 16
CHUNK = 80
EDGES_PER_TILE = N_EDGES // NUM_SUBCORES
CHUNKS_PER_TILE = EDGES_PER_TILE // CHUNK
PADN = 10240
ROWS_PER_TILE = PADN // NUM_SUBCORES


def _sc_aggregate_body(xflat, src3, dstf, ewf, out, sidx2,
                       didx0, didx1, didx2b, ewc0, ewc1, ewc2,
                       rows0, rows1, rows2,
                       gsem0, gsem1, gsem2, esem0, esem1, esem2,
                       ssem0, ssem1, ssem2, acc):
    cid = lax.axis_index("c")
    sid = lax.axis_index("s")
    ebase = sid * EDGES_PER_TILE

    sc = pltpu.async_copy(src3.at[sid], sidx2, gsem0)

    zv = jnp.zeros((LANES,), jnp.float32)

    def _zrow(e, carry):
        for k in range(HALF // LANES):
            rows0[e, pl.ds(k * LANES, LANES)] = zv
        return carry

    lax.fori_loop(0, CHUNK, _zrow, 0)
    for j in range(ROWS_PER_TILE // CHUNK):
        pltpu.sync_copy(
            rows0, acc.at[pl.ds(sid * ROWS_PER_TILE + j * CHUNK, CHUNK)])

    sc.wait()

    cbase = cid * N_NODES

    def _adjust(j, carry):
        for k in range(CHUNK // LANES):
            sl = pl.ds(k * LANES, LANES)
            sidx2[j, sl] = sidx2[j, sl] + cbase
        return carry

    lax.fori_loop(0, CHUNKS_PER_TILE, _adjust, 0)
    plsc.subcore_barrier()

    def _drain(buf_rows, buf_di, ssem):
        pltpu.make_async_copy(buf_rows, acc.at[buf_di], ssem).wait()

    def _issue(j, buf_rows, buf_ew, buf_di, gsem, esem):
        pltpu.async_copy(ewf.at[pl.ds(ebase + j * CHUNK, CHUNK)], buf_ew,
                         esem)
        pltpu.async_copy(dstf.at[pl.ds(ebase + j * CHUNK, CHUNK)], buf_di,
                         esem)
        pltpu.async_copy(xflat.at[sidx2.at[j]], buf_rows, gsem)

    def _process(j, buf_rows, buf_ew, buf_di, gsem, esem, ssem):
        pltpu.make_async_copy(ewf.at[pl.ds(ebase + j * CHUNK, CHUNK)],
                              buf_ew, esem).wait()
        pltpu.make_async_copy(dstf.at[pl.ds(ebase + j * CHUNK, CHUNK)],
                              buf_di, esem).wait()
        pltpu.make_async_copy(xflat.at[sidx2.at[j]], buf_rows, gsem).wait()

        def _group(gidx, c2):
            wv = buf_ew[pl.ds(gidx * LANES, LANES)]
            for t in range(LANES):
                w = wv.at[jnp.full((LANES,), t, jnp.int32)].get(
                    mode="promise_in_bounds")
                e = gidx * LANES + t
                for k in range(HALF // LANES):
                    sl = pl.ds(k * LANES, LANES)
                    buf_rows[e, sl] = buf_rows[e, sl] * w
            return c2

        lax.fori_loop(0, CHUNK // LANES, _group, 0)

        pltpu.async_copy(buf_rows, acc.at[buf_di], ssem, add=True)

    sets = ((rows0, ewc0, didx0, gsem0, esem0, ssem0),
            (rows1, ewc1, didx1, gsem1, esem1, ssem1),
            (rows2, ewc2, didx2b, gsem2, esem2, ssem2))

    def _issue_set(j, s):
        _issue(j, s[0], s[1], s[2], s[3], s[4])

    def _process_set(j, s):
        _process(j, *s)

    def _drain_set(s):
        _drain(s[0], s[2], s[5])

    _issue_set(0, sets[0])

    def _triple(p, carry):
        j0 = p * 3
        for t in range(3):
            j = j0 + t
            nxt = sets[(t + 1) % 3]
            if t == 2:
                _drain_set(nxt)
            else:
                @pl.when(p > 0)
                def _():
                    _drain_set(nxt)
            _issue_set(j + 1, nxt)
            _process_set(j, sets[t])
        return carry

    lax.fori_loop(0, (CHUNKS_PER_TILE - 2) // 3, _triple, 0)
    _drain_set(sets[1])
    _issue_set(CHUNKS_PER_TILE - 1, sets[1])
    _process_set(CHUNKS_PER_TILE - 2, sets[0])
    _process_set(CHUNKS_PER_TILE - 1, sets[1])
    for s in sets:
        _drain_set(s)
    plsc.subcore_barrier()

    r0 = sid * ROWS_PER_TILE
    pltpu.sync_copy(acc.at[pl.ds(r0, ROWS_PER_TILE)],
                    out.at[pl.ds(cid * PADN + r0, ROWS_PER_TILE)])


def _sc_aggregate(xflat, src, dst, ew):
    src3 = src.reshape(NUM_SUBCORES, CHUNKS_PER_TILE, CHUNK)
    kern = pl.kernel(
        _sc_aggregate_body,
        mesh=plsc.VectorSubcoreMesh(core_axis_name="c", subcore_axis_name="s"),
        out_type=jax.ShapeDtypeStruct((NUM_CORES * PADN, HALF),
                                      jnp.float32),
        scratch_types=[
            pltpu.VMEM((CHUNKS_PER_TILE, CHUNK), jnp.int32),
            pltpu.VMEM((CHUNK,), jnp.int32),
            pltpu.VMEM((CHUNK,), jnp.int32),
            pltpu.VMEM((CHUNK,), jnp.int32),
            pltpu.VMEM((CHUNK,), jnp.float32),
            pltpu.VMEM((CHUNK,), jnp.float32),
            pltpu.VMEM((CHUNK,), jnp.float32),
            pltpu.VMEM((CHUNK, HALF), jnp.float32),
            pltpu.VMEM((CHUNK, HALF), jnp.float32),
            pltpu.VMEM((CHUNK, HALF), jnp.float32),
            pltpu.SemaphoreType.DMA,
            pltpu.SemaphoreType.DMA,
            pltpu.SemaphoreType.DMA,
            pltpu.SemaphoreType.DMA,
            pltpu.SemaphoreType.DMA,
            pltpu.SemaphoreType.DMA,
            pltpu.SemaphoreType.DMA,
            pltpu.SemaphoreType.DMA,
            pltpu.SemaphoreType.DMA,
            pltpu.VMEM_SHARED((PADN, HALF), jnp.float32),
        ],
    )
    return kern(xflat, src3, dst, ew)


BN = 1000
NB = N_NODES // BN


def _dense_stats(agg_ref, h_ref, Wr, br, Wo, b, S_ref, sums_ref):
    agg = jnp.concatenate([agg_ref[0], agg_ref[1]], axis=1)
    h = jnp.concatenate([h_ref[0], h_ref[1]], axis=1)
    s = (jnp.dot(agg, Wr[...], preferred_element_type=jnp.float32)
         + jnp.dot(h, Wo[...], preferred_element_type=jnp.float32)
         + br[...])
    S_ref[pl.ds(b * BN, BN), :] = s
    s0 = jnp.sum(s, axis=0, keepdims=True)
    s1 = jnp.sum(s * s, axis=0, keepdims=True)

    @pl.when(b == 0)
    def _():
        sums_ref[0:1] = s0
        sums_ref[1:2] = s1

    @pl.when(b > 0)
    def _():
        sums_ref[0:1] = sums_ref[0:1] + s0
        sums_ref[1:2] = sums_ref[1:2] + s1


def _bn_relu(S_ref, sums_ref, g, be, b):
    m = sums_ref[0:1] / N_NODES
    v = sums_ref[1:2] / N_NODES - m * m
    s = S_ref[pl.ds(b * BN, BN), :]
    y = (s - m) * jax.lax.rsqrt(v + 1e-5) * g[...] + be[...]
    return jnp.maximum(y, 0.0)


def _dense_body(agg_ref, h_ref, Wr, br, Wo, g, be, out_ref, S_ref, sums_ref):
    p = pl.program_id(0)
    b = pl.program_id(1)

    @pl.when(p == 0)
    def _():
        _dense_stats(agg_ref, h_ref, Wr, br, Wo, b, S_ref, sums_ref)

    @pl.when(p == 1)
    def _():
        y = _bn_relu(S_ref, sums_ref, g, be, b)
        out_ref[0] = y[:, :HALF]
        out_ref[1] = y[:, HALF:]


def _tc_dense(aggf, hf, Wr, br, Wo, g, be):
    agg3 = aggf.reshape(NUM_CORES, PADN, HALF)
    h3 = hf.reshape(NUM_CORES, N_NODES, HALF)
    out = pl.pallas_call(
        _dense_body,
        grid=(2, NB),
        in_specs=[
            pl.BlockSpec((NUM_CORES, BN, HALF),
                         lambda p, b: (0, b * (1 - p), 0)),
            pl.BlockSpec((NUM_CORES, BN, HALF),
                         lambda p, b: (0, b * (1 - p), 0)),
            pl.BlockSpec((FEAT, FEAT), lambda p, b: (0, 0)),
            pl.BlockSpec((1, FEAT), lambda p, b: (0, 0)),
            pl.BlockSpec((FEAT, FEAT), lambda p, b: (0, 0)),
            pl.BlockSpec((1, FEAT), lambda p, b: (0, 0)),
            pl.BlockSpec((1, FEAT), lambda p, b: (0, 0)),
        ],
        out_specs=pl.BlockSpec((NUM_CORES, BN, HALF),
                               lambda p, b: (0, b * p, 0)),
        out_shape=jax.ShapeDtypeStruct((NUM_CORES, N_NODES, HALF),
                                       jnp.float32),
        scratch_shapes=[pltpu.VMEM((N_NODES, FEAT), jnp.float32),
                        pltpu.VMEM((2, FEAT), jnp.float32)],
    )(agg3, h3, Wr, br.reshape(1, FEAT), Wo, g.reshape(1, FEAT),
      be.reshape(1, FEAT))
    return out.reshape(NUM_CORES * N_NODES, HALF)


def _final_body(agg_ref, h_ref, Wr, br, Wo, g, be, batch_ref, W1, b1, W2, b2,
                out_ref, S_ref, sums_ref, pool_ref):
    p = pl.program_id(0)
    b = pl.program_id(1)

    @pl.when(jnp.logical_and(p == 0, b == 0))
    def _():
        pool_ref[...] = jnp.full((N_GRAPHS, FEAT), -jnp.inf, jnp.float32)

    @pl.when(p == 0)
    def _():
        _dense_stats(agg_ref, h_ref, Wr, br, Wo, b, S_ref, sums_ref)

    @pl.when(p == 1)
    def _():
        y = _bn_relu(S_ref, sums_ref, g, be, b)
        bvec = batch_ref[...]
        gmin = jnp.min(bvec)
        gmax = jnp.max(bvec)

        def _pool(gi, carry):
            mg = jnp.max(jnp.where(bvec == gi, y, -jnp.inf), axis=0)
            pool_ref[pl.ds(gi, 1), :] = jnp.maximum(
                pool_ref[pl.ds(gi, 1), :], mg[None, :])
            return carry

        lax.fori_loop(gmin, gmax + 1, _pool, 0)

        @pl.when(b == NB - 1)
        def _():
            pm = pool_ref[...]
            o1 = jnp.maximum(
                jnp.dot(pm, W1[...], preferred_element_type=jnp.float32)
                + b1[...], 0.0)
            o2 = (jnp.dot(o1, W2[...], preferred_element_type=jnp.float32)
                  + b2[...])
            out_ref[...] = 1.0 / (1.0 + jnp.exp(-o2))


def _tc_final(aggf, hf, Wr, br, Wo, g, be, batch, W1, b1, W2, b2):
    agg3 = aggf.reshape(NUM_CORES, PADN, HALF)
    h3 = hf.reshape(NUM_CORES, N_NODES, HALF)
    return pl.pallas_call(
        _final_body,
        grid=(2, NB),
        in_specs=[
            pl.BlockSpec((NUM_CORES, BN, HALF),
                         lambda p, b: (0, b * (1 - p), 0)),
            pl.BlockSpec((NUM_CORES, BN, HALF),
                         lambda p, b: (0, b * (1 - p), 0)),
            pl.BlockSpec((FEAT, FEAT), lambda p, b: (0, 0)),
            pl.BlockSpec((1, FEAT), lambda p, b: (0, 0)),
            pl.BlockSpec((FEAT, FEAT), lambda p, b: (0, 0)),
            pl.BlockSpec((1, FEAT), lambda p, b: (0, 0)),
            pl.BlockSpec((1, FEAT), lambda p, b: (0, 0)),
            pl.BlockSpec((BN, 1), lambda p, b: (b * p, 0)),
            pl.BlockSpec((FEAT, 16), lambda p, b: (0, 0)),
            pl.BlockSpec((1, 16), lambda p, b: (0, 0)),
            pl.BlockSpec((16, 1), lambda p, b: (0, 0)),
            pl.BlockSpec((1, 1), lambda p, b: (0, 0)),
        ],
        out_specs=pl.BlockSpec((N_GRAPHS, 1), lambda p, b: (0, 0)),
        out_shape=jax.ShapeDtypeStruct((N_GRAPHS, 1), jnp.float32),
        scratch_shapes=[pltpu.VMEM((N_NODES, FEAT), jnp.float32),
                        pltpu.VMEM((2, FEAT), jnp.float32),
                        pltpu.VMEM((N_GRAPHS, FEAT), jnp.float32)],
    )(agg3, h3, Wr, br.reshape(1, FEAT), Wo, g.reshape(1, FEAT),
      be.reshape(1, FEAT), batch.reshape(N_NODES, 1), W1,
      b1.reshape(1, 16), W2, b2.reshape(1, 1))


def kernel(x, edge_index, edge_attr, batch, Wr1, br1, Wo1, Wr2, br2, Wo2,
           Wr3, br3, Wo3, g1, be1, g2, be2, g3, be3, W1, b1, W2, b2):
    src = edge_index[0]
    dst = edge_index[1]
    xflat = jnp.concatenate([x[:, :HALF], x[:, HALF:]], axis=0)

    agg1 = _sc_aggregate(xflat, src, dst, edge_attr)
    h1 = _tc_dense(agg1, xflat, Wr1, br1, Wo1, g1, be1)
    agg2 = _sc_aggregate(h1, src, dst, edge_attr)
    h2 = _tc_dense(agg2, h1, Wr2, br2, Wo2, g2, be2)
    agg3 = _sc_aggregate(h2, src, dst, edge_attr)
    return _tc_final(agg3, h2, Wr3, br3, Wo3, g3, be3, batch, W1, b1, W2, b2)


# speedup vs baseline: 1.0049x; 1.0049x over previous
import jax
import jax.numpy as jnp
from jax import lax
from jax.experimental import pallas as pl
from jax.experimental.pallas import tpu as pltpu
from jax.experimental.pallas import tpu_sc as plsc

N_NODES = 10000
N_EDGES = 160000
FEAT = 256
HALF = 128
N_GRAPHS = 64

NUM_CORES = 2
NUM_SUBCORES = 16
LANES = 16
CHUNK = 80
EDGES_PER_TILE = N_EDGES // NUM_SUBCORES
CHUNKS_PER_TILE = EDGES_PER_TILE // CHUNK
PADN = 10240
ROWS_PER_TILE = PADN // NUM_SUBCORES


def _sc_aggregate_body(xflat, src3, dstf, ewf, out, sidx2,
                       didx0, didx1, didx2b, ewc0, ewc1, ewc2,
                       rows0, rows1, rows2,
                       gsem0, gsem1, gsem2, esem0, esem1, esem2,
                       ssem0, ssem1, ssem2, acc):
    cid = lax.axis_index("c")
    sid = lax.axis_index("s")
    ebase = sid * EDGES_PER_TILE

    sc = pltpu.async_copy(src3.at[sid], sidx2, gsem0)

    zv = jnp.zeros((LANES,), jnp.float32)

    def _zrow(e, carry):
        for k in range(HALF // LANES):
            rows0[e, pl.ds(k * LANES, LANES)] = zv
        return carry

    lax.fori_loop(0, CHUNK, _zrow, 0)
    for j in range(ROWS_PER_TILE // CHUNK):
        pltpu.sync_copy(
            rows0, acc.at[pl.ds(sid * ROWS_PER_TILE + j * CHUNK, CHUNK)])

    sc.wait()

    cbase = cid * N_NODES

    def _adjust(j, carry):
        for k in range(CHUNK // LANES):
            sl = pl.ds(k * LANES, LANES)
            sidx2[j, sl] = sidx2[j, sl] + cbase
        return carry

    lax.fori_loop(0, CHUNKS_PER_TILE, _adjust, 0)
    plsc.subcore_barrier()

    def _drain(buf_rows, buf_di, ssem):
        pltpu.make_async_copy(buf_rows, acc.at[buf_di], ssem).wait()

    def _issue(j, buf_rows, buf_ew, buf_di, gsem, esem):
        pltpu.async_copy(ewf.at[pl.ds(ebase + j * CHUNK, CHUNK)], buf_ew,
                         esem)
        pltpu.async_copy(dstf.at[pl.ds(ebase + j * CHUNK, CHUNK)], buf_di,
                         esem)
        pltpu.async_copy(xflat.at[sidx2.at[j]], buf_rows, gsem)

    def _process(j, buf_rows, buf_ew, buf_di, gsem, esem, ssem):
        pltpu.make_async_copy(ewf.at[pl.ds(ebase + j * CHUNK, CHUNK)],
                              buf_ew, esem).wait()
        pltpu.make_async_copy(dstf.at[pl.ds(ebase + j * CHUNK, CHUNK)],
                              buf_di, esem).wait()
        pltpu.make_async_copy(xflat.at[sidx2.at[j]], buf_rows, gsem).wait()

        def _group(gidx, c2):
            wv = buf_ew[pl.ds(gidx * LANES, LANES)]
            for t in range(LANES):
                w = wv.at[jnp.full((LANES,), t, jnp.int32)].get(
                    mode="promise_in_bounds")
                e = gidx * LANES + t
                for k in range(HALF // LANES):
                    sl = pl.ds(k * LANES, LANES)
                    buf_rows[e, sl] = buf_rows[e, sl] * w
            return c2

        lax.fori_loop(0, CHUNK // LANES, _group, 0)

        pltpu.async_copy(buf_rows, acc.at[buf_di], ssem, add=True)

    sets = ((rows0, ewc0, didx0, gsem0, esem0, ssem0),
            (rows1, ewc1, didx1, gsem1, esem1, ssem1),
            (rows2, ewc2, didx2b, gsem2, esem2, ssem2))

    def _issue_set(j, s):
        _issue(j, s[0], s[1], s[2], s[3], s[4])

    def _process_set(j, s):
        _process(j, *s)

    def _drain_set(s):
        _drain(s[0], s[2], s[5])

    _issue_set(0, sets[0])

    def _triple(p, carry):
        j0 = p * 3
        for t in range(3):
            j = j0 + t
            nxt = sets[(t + 1) % 3]
            if t == 2:
                _drain_set(nxt)
            else:
                @pl.when(p > 0)
                def _():
                    _drain_set(nxt)
            _issue_set(j + 1, nxt)
            _process_set(j, sets[t])
        return carry

    lax.fori_loop(0, (CHUNKS_PER_TILE - 2) // 3, _triple, 0)
    _drain_set(sets[1])
    _issue_set(CHUNKS_PER_TILE - 1, sets[1])
    _process_set(CHUNKS_PER_TILE - 2, sets[0])
    _process_set(CHUNKS_PER_TILE - 1, sets[1])
    for s in sets:
        _drain_set(s)
    plsc.subcore_barrier()

    r0 = sid * ROWS_PER_TILE
    pltpu.sync_copy(acc.at[pl.ds(r0, ROWS_PER_TILE)],
                    out.at[pl.ds(cid * PADN + r0, ROWS_PER_TILE)])


def _sc_aggregate(xflat, src, dst, ew):
    src3 = src.reshape(NUM_SUBCORES, CHUNKS_PER_TILE, CHUNK)
    kern = pl.kernel(
        _sc_aggregate_body,
        mesh=plsc.VectorSubcoreMesh(core_axis_name="c", subcore_axis_name="s"),
        out_type=jax.ShapeDtypeStruct((NUM_CORES * PADN, HALF),
                                      jnp.float32),
        scratch_types=[
            pltpu.VMEM((CHUNKS_PER_TILE, CHUNK), jnp.int32),
            pltpu.VMEM((CHUNK,), jnp.int32),
            pltpu.VMEM((CHUNK,), jnp.int32),
            pltpu.VMEM((CHUNK,), jnp.int32),
            pltpu.VMEM((CHUNK,), jnp.float32),
            pltpu.VMEM((CHUNK,), jnp.float32),
            pltpu.VMEM((CHUNK,), jnp.float32),
            pltpu.VMEM((CHUNK, HALF), jnp.float32),
            pltpu.VMEM((CHUNK, HALF), jnp.float32),
            pltpu.VMEM((CHUNK, HALF), jnp.float32),
            pltpu.SemaphoreType.DMA,
            pltpu.SemaphoreType.DMA,
            pltpu.SemaphoreType.DMA,
            pltpu.SemaphoreType.DMA,
            pltpu.SemaphoreType.DMA,
            pltpu.SemaphoreType.DMA,
            pltpu.SemaphoreType.DMA,
            pltpu.SemaphoreType.DMA,
            pltpu.SemaphoreType.DMA,
            pltpu.VMEM_SHARED((PADN, HALF), jnp.float32),
        ],
    )
    return kern(xflat, src3, dst, ew)


BN = 2000
NB = N_NODES // BN


def _dense_stats(agg_ref, h_ref, Wr, br, Wo, b, S_ref, sums_ref):
    agg = jnp.concatenate([agg_ref[0], agg_ref[1]], axis=1)
    h = jnp.concatenate([h_ref[0], h_ref[1]], axis=1)
    s = (jnp.dot(agg, Wr[...], preferred_element_type=jnp.float32)
         + jnp.dot(h, Wo[...], preferred_element_type=jnp.float32)
         + br[...])
    S_ref[pl.ds(b * BN, BN), :] = s
    s0 = jnp.sum(s, axis=0, keepdims=True)
    s1 = jnp.sum(s * s, axis=0, keepdims=True)

    @pl.when(b == 0)
    def _():
        sums_ref[0:1] = s0
        sums_ref[1:2] = s1

    @pl.when(b > 0)
    def _():
        sums_ref[0:1] = sums_ref[0:1] + s0
        sums_ref[1:2] = sums_ref[1:2] + s1


def _bn_relu(S_ref, sums_ref, g, be, b):
    m = sums_ref[0:1] / N_NODES
    v = sums_ref[1:2] / N_NODES - m * m
    s = S_ref[pl.ds(b * BN, BN), :]
    y = (s - m) * jax.lax.rsqrt(v + 1e-5) * g[...] + be[...]
    return jnp.maximum(y, 0.0)


def _dense_body(agg_ref, h_ref, Wr, br, Wo, g, be, out_ref, S_ref, sums_ref):
    p = pl.program_id(0)
    b = pl.program_id(1)

    @pl.when(p == 0)
    def _():
        _dense_stats(agg_ref, h_ref, Wr, br, Wo, b, S_ref, sums_ref)

    @pl.when(p == 1)
    def _():
        y = _bn_relu(S_ref, sums_ref, g, be, b)
        out_ref[0] = y[:, :HALF]
        out_ref[1] = y[:, HALF:]


def _tc_dense(aggf, hf, Wr, br, Wo, g, be):
    agg3 = aggf.reshape(NUM_CORES, PADN, HALF)
    h3 = hf.reshape(NUM_CORES, N_NODES, HALF)
    out = pl.pallas_call(
        _dense_body,
        grid=(2, NB),
        in_specs=[
            pl.BlockSpec((NUM_CORES, BN, HALF),
                         lambda p, b: (0, b * (1 - p), 0)),
            pl.BlockSpec((NUM_CORES, BN, HALF),
                         lambda p, b: (0, b * (1 - p), 0)),
            pl.BlockSpec((FEAT, FEAT), lambda p, b: (0, 0)),
            pl.BlockSpec((1, FEAT), lambda p, b: (0, 0)),
            pl.BlockSpec((FEAT, FEAT), lambda p, b: (0, 0)),
            pl.BlockSpec((1, FEAT), lambda p, b: (0, 0)),
            pl.BlockSpec((1, FEAT), lambda p, b: (0, 0)),
        ],
        out_specs=pl.BlockSpec((NUM_CORES, BN, HALF),
                               lambda p, b: (0, b * p, 0)),
        out_shape=jax.ShapeDtypeStruct((NUM_CORES, N_NODES, HALF),
                                       jnp.float32),
        scratch_shapes=[pltpu.VMEM((N_NODES, FEAT), jnp.float32),
                        pltpu.VMEM((2, FEAT), jnp.float32)],
    )(agg3, h3, Wr, br.reshape(1, FEAT), Wo, g.reshape(1, FEAT),
      be.reshape(1, FEAT))
    return out.reshape(NUM_CORES * N_NODES, HALF)


def _final_body(agg_ref, h_ref, Wr, br, Wo, g, be, batch_ref, W1, b1, W2, b2,
                out_ref, S_ref, sums_ref, pool_ref):
    p = pl.program_id(0)
    b = pl.program_id(1)

    @pl.when(jnp.logical_and(p == 0, b == 0))
    def _():
        pool_ref[...] = jnp.full((N_GRAPHS, FEAT), -jnp.inf, jnp.float32)

    @pl.when(p == 0)
    def _():
        _dense_stats(agg_ref, h_ref, Wr, br, Wo, b, S_ref, sums_ref)

    @pl.when(p == 1)
    def _():
        y = _bn_relu(S_ref, sums_ref, g, be, b)
        bvec = batch_ref[...]
        gmin = jnp.min(bvec)
        gmax = jnp.max(bvec)

        def _pool(gi, carry):
            mg = jnp.max(jnp.where(bvec == gi, y, -jnp.inf), axis=0)
            pool_ref[pl.ds(gi, 1), :] = jnp.maximum(
                pool_ref[pl.ds(gi, 1), :], mg[None, :])
            return carry

        lax.fori_loop(gmin, gmax + 1, _pool, 0)

        @pl.when(b == NB - 1)
        def _():
            pm = pool_ref[...]
            o1 = jnp.maximum(
                jnp.dot(pm, W1[...], preferred_element_type=jnp.float32)
                + b1[...], 0.0)
            o2 = (jnp.dot(o1, W2[...], preferred_element_type=jnp.float32)
                  + b2[...])
            out_ref[...] = 1.0 / (1.0 + jnp.exp(-o2))


def _tc_final(aggf, hf, Wr, br, Wo, g, be, batch, W1, b1, W2, b2):
    agg3 = aggf.reshape(NUM_CORES, PADN, HALF)
    h3 = hf.reshape(NUM_CORES, N_NODES, HALF)
    return pl.pallas_call(
        _final_body,
        grid=(2, NB),
        in_specs=[
            pl.BlockSpec((NUM_CORES, BN, HALF),
                         lambda p, b: (0, b * (1 - p), 0)),
            pl.BlockSpec((NUM_CORES, BN, HALF),
                         lambda p, b: (0, b * (1 - p), 0)),
            pl.BlockSpec((FEAT, FEAT), lambda p, b: (0, 0)),
            pl.BlockSpec((1, FEAT), lambda p, b: (0, 0)),
            pl.BlockSpec((FEAT, FEAT), lambda p, b: (0, 0)),
            pl.BlockSpec((1, FEAT), lambda p, b: (0, 0)),
            pl.BlockSpec((1, FEAT), lambda p, b: (0, 0)),
            pl.BlockSpec((BN, 1), lambda p, b: (b * p, 0)),
            pl.BlockSpec((FEAT, 16), lambda p, b: (0, 0)),
            pl.BlockSpec((1, 16), lambda p, b: (0, 0)),
            pl.BlockSpec((16, 1), lambda p, b: (0, 0)),
            pl.BlockSpec((1, 1), lambda p, b: (0, 0)),
        ],
        out_specs=pl.BlockSpec((N_GRAPHS, 1), lambda p, b: (0, 0)),
        out_shape=jax.ShapeDtypeStruct((N_GRAPHS, 1), jnp.float32),
        scratch_shapes=[pltpu.VMEM((N_NODES, FEAT), jnp.float32),
                        pltpu.VMEM((2, FEAT), jnp.float32),
                        pltpu.VMEM((N_GRAPHS, FEAT), jnp.float32)],
    )(agg3, h3, Wr, br.reshape(1, FEAT), Wo, g.reshape(1, FEAT),
      be.reshape(1, FEAT), batch.reshape(N_NODES, 1), W1,
      b1.reshape(1, 16), W2, b2.reshape(1, 1))


def kernel(x, edge_index, edge_attr, batch, Wr1, br1, Wo1, Wr2, br2, Wo2,
           Wr3, br3, Wo3, g1, be1, g2, be2, g3, be3, W1, b1, W2, b2):
    src = edge_index[0]
    dst = edge_index[1]
    xflat = jnp.concatenate([x[:, :HALF], x[:, HALF:]], axis=0)

    agg1 = _sc_aggregate(xflat, src, dst, edge_attr)
    h1 = _tc_dense(agg1, xflat, Wr1, br1, Wo1, g1, be1)
    agg2 = _sc_aggregate(h1, src, dst, edge_attr)
    h2 = _tc_dense(agg2, h1, Wr2, br2, Wo2, g2, be2)
    agg3 = _sc_aggregate(h2, src, dst, edge_attr)
    return _tc_final(agg3, h2, Wr3, br3, Wo3, g3, be3, batch, W1, b1, W2, b2)
